# baseline (speedup 1.0000x reference)
import functools

import jax
import jax.numpy as jnp
from jax import lax
from jax.experimental import pallas as pl
from jax.experimental.pallas import tpu as pltpu
from jax.experimental.pallas import tpu_sc as plsc


def _make_sc_broadcast(batch, n_sents, emb, dtype):
    info = plsc.get_sparse_core_info()
    nc, ns = info.num_cores, info.num_subcores
    nw = nc * ns
    b_per_w = batch // nw
    mesh = plsc.VectorSubcoreMesh(core_axis_name="c", subcore_axis_name="s")

    @functools.partial(
        pl.kernel,
        mesh=mesh,
        out_type=jax.ShapeDtypeStruct((batch, n_sents, emb), dtype),
        scratch_types=[
            pltpu.VMEM((n_sents, emb), dtype),
            pltpu.SemaphoreType.DMA,
        ],
    )
    def k(tbl_hbm, out_hbm, tbl_v, sem):
        wid = lax.axis_index("s") * nc + lax.axis_index("c")
        base = wid * b_per_w
        pltpu.sync_copy(tbl_hbm.at[pl.ds(0, n_sents)], tbl_v)
        copies = [
            pltpu.async_copy(tbl_v, out_hbm.at[base + i], sem)
            for i in range(b_per_w)
        ]
        for c in copies:
            c.wait()

    return k


def kernel(top_vecs, tok_struct_vec, sent_struct_vec, pos_table):
    batch, n_sents = top_vecs.shape[0], top_vecs.shape[1]
    emb = pos_table.shape[1]
    sc_fn = _make_sc_broadcast(batch, n_sents, emb, pos_table.dtype)
    return sc_fn(pos_table)
